# baseline (speedup 1.0000x reference)
import functools

import jax
import jax.numpy as jnp
from jax import lax
from jax.experimental import pallas as pl
from jax.experimental.pallas import tpu as pltpu
from jax.experimental.pallas import tpu_sc as plsc


def _lane_perm_idx():
    i = lax.iota(jnp.int32, 16)
    return jnp.where(i == 1, i + 1, jnp.where(i == 2, i - 1, i))


_GATHER_DNUMS = lax.GatherDimensionNumbers(
    offset_dims=(), collapsed_slice_dims=(0,), start_index_map=(0,)
)


def _make_sc_kernel():
    mesh = plsc.VectorSubcoreMesh(core_axis_name="c", subcore_axis_name="s")

    @functools.partial(
        pl.kernel,
        mesh=mesh,
        out_type=jax.ShapeDtypeStruct((64, 16), jnp.float32),
        scratch_types=[
            pltpu.VMEM((64, 16), jnp.float32),
            pltpu.VMEM((64, 16), jnp.float32),
        ],
    )
    def k(xt_hbm, out_hbm, slab_v, out_v):
        cid = lax.axis_index("c")
        sid = lax.axis_index("s")

        @pl.when(jnp.logical_and(cid == 0, sid == 0))
        def _():
            for c in range(64):
                pltpu.sync_copy(xt_hbm.at[c, pl.ds(0, 16)], slab_v.at[c])
            perm = _lane_perm_idx().reshape(16, 1)
            for c in range(64):
                vec = slab_v[c, pl.ds(0, 16)]
                shuf = lax.gather(
                    vec,
                    perm,
                    dimension_numbers=_GATHER_DNUMS,
                    slice_sizes=(1,),
                    mode=lax.GatherScatterMode.PROMISE_IN_BOUNDS,
                )
                out_v[c, pl.ds(0, 16)] = shuf
            pltpu.sync_copy(out_v, out_hbm)

    return k


_sc_kernel = _make_sc_kernel()


def kernel(x):
    xt = x.T
    out_t = _sc_kernel(xt)
    return out_t[:, :4].T.reshape(1, 2, 2, 64)


# speedup vs baseline: 2.2068x; 2.2068x over previous
import functools

import jax
import jax.numpy as jnp
from jax import lax
from jax.experimental import pallas as pl
from jax.experimental.pallas import tpu as pltpu
from jax.experimental.pallas import tpu_sc as plsc


def _lane_perm_idx():
    i = lax.iota(jnp.int32, 16)
    return jnp.where(i == 1, i + 1, jnp.where(i == 2, i - 1, i))


_GATHER_DNUMS = lax.GatherDimensionNumbers(
    offset_dims=(), collapsed_slice_dims=(0,), start_index_map=(0,)
)


def _make_sc_kernel():
    mesh = plsc.VectorSubcoreMesh(core_axis_name="c", subcore_axis_name="s")

    @functools.partial(
        pl.kernel,
        mesh=mesh,
        out_type=jax.ShapeDtypeStruct((64, 16), jnp.float32),
        scratch_types=[
            pltpu.VMEM((8, 16), jnp.float32),
            pltpu.VMEM((8, 16), jnp.float32),
            pltpu.SemaphoreType.DMA((8,)),
        ],
    )
    def k(xt_hbm, out_hbm, slab_v, out_v, sems):
        cid = lax.axis_index("c")
        sid = lax.axis_index("s")
        wid = cid * 16 + sid

        @pl.when(wid < 8)
        def _():
            base = wid * 8
            copies = []
            for r in range(8):
                copies.append(
                    pltpu.async_copy(
                        xt_hbm.at[base + r, pl.ds(0, 16)],
                        slab_v.at[r],
                        sems.at[r],
                    )
                )
            for c in copies:
                c.wait()
            perm = _lane_perm_idx().reshape(16, 1)
            for r in range(8):
                vec = slab_v[r, pl.ds(0, 16)]
                shuf = lax.gather(
                    vec,
                    perm,
                    dimension_numbers=_GATHER_DNUMS,
                    slice_sizes=(1,),
                    mode=lax.GatherScatterMode.PROMISE_IN_BOUNDS,
                )
                out_v[r, pl.ds(0, 16)] = shuf
            pltpu.sync_copy(out_v, out_hbm.at[pl.ds(base, 8)])

    return k


_sc_kernel = _make_sc_kernel()


def kernel(x):
    xt = x.T
    out_t = _sc_kernel(xt)
    return out_t[:, :4].T.reshape(1, 2, 2, 64)


# speedup vs baseline: 32.2173x; 14.5989x over previous
import jax
import jax.numpy as jnp
from jax.experimental import pallas as pl


def _gather_kernel(xt_ref, o_ref):
    t = jnp.transpose(xt_ref[...])
    o_ref[...] = jnp.concatenate(
        [t[0:1, :], t[2:3, :], t[1:2, :], t[3:4, :]], axis=0
    )


def kernel(x):
    xt = x.T
    out = pl.pallas_call(
        _gather_kernel,
        out_shape=jax.ShapeDtypeStruct((4, 64), jnp.float32),
        grid=(1,),
        in_specs=[pl.BlockSpec((64, 128), lambda i: (0, 0))],
        out_specs=pl.BlockSpec((4, 64), lambda i: (0, 0)),
    )(xt)
    return out.reshape(1, 2, 2, 64)
